# Initial kernel scaffold; baseline (speedup 1.0000x reference)
import jax
import jax.numpy as jnp
from jax.experimental import pallas as pl
from jax.experimental.pallas import tpu as pltpu

NTOK = 1000
SPLIT = 500
SENT = 1000
TEMP = 65.0
N = 1024
H = 64
WPAD = 1024
BN = 16
BW = 128
NCHUNK = WPAD // BW


def _loss_kernel(hid_ref, tgt_ref, emb_ref, wih_ref, whh_ref, out_ref,
                 at_ref):
    i = pl.program_id(0)

    @pl.when(i == 0)
    def _prep():
        a = jnp.dot(emb_ref[...], wih_ref[...],
                    preferred_element_type=jnp.float32)
        at_ref[...] = a.T
        out_ref[...] = jnp.zeros_like(out_ref)

    h_blk = hid_ref[...]
    b_blk = jnp.dot(h_blk, whh_ref[...],
                    preferred_element_type=jnp.float32)
    tgt = tgt_ref[...]

    head_s = jnp.zeros((BN, 1), jnp.float32)
    tail_s = jnp.zeros((BN, 1), jnp.float32)
    tgt_k = jnp.zeros((BN, 1), jnp.float32)
    sent_k = jnp.zeros((BN, 1), jnp.float32)

    b3 = b_blk[:, :, None]
    h3 = h_blk[:, :, None]

    for c in range(NCHUNK):
        at_c = at_ref[:, c * BW:(c + 1) * BW]
        wids = c * BW + jax.lax.broadcasted_iota(
            jnp.int32, (1, BW), 1)
        t = jnp.tanh(at_c[None, :, :] + b3)
        diff = h3 - t
        d = jnp.sum(diff * diff, axis=1)
        k = TEMP / (1.0 + d)
        e = jnp.exp(k - TEMP)
        headmask = (wids < SPLIT) | (wids == SENT)
        tailmask = (wids >= SPLIT) & (wids < NTOK)
        head_s += jnp.sum(jnp.where(headmask, e, 0.0), axis=1,
                          keepdims=True)
        tail_s += jnp.sum(jnp.where(tailmask, e, 0.0), axis=1,
                          keepdims=True)
        tgt_k += jnp.sum(jnp.where(wids == tgt, k, 0.0), axis=1,
                         keepdims=True)
        sent_k += jnp.sum(jnp.where(wids == SENT, k, 0.0), axis=1,
                          keepdims=True)

    lse_head = jnp.log(head_s) + TEMP
    lse_tail = jnp.log(tail_s) + TEMP
    is_tail = (tgt >= SPLIT).astype(jnp.float32)
    loss = lse_head - tgt_k + is_tail * (lse_tail - sent_k)
    out_ref[0, 0] += jnp.sum(loss) * (1.0 / N)


@jax.jit
def kernel(hiddens, targets, emb, W_ih, W_hh):
    emb_pad = jnp.zeros((WPAD, H), jnp.float32).at[:emb.shape[0]].set(emb)
    tgt2d = targets.reshape(N, 1)
    out = pl.pallas_call(
        _loss_kernel,
        grid=(N // BN,),
        in_specs=[
            pl.BlockSpec((BN, H), lambda i: (i, 0)),
            pl.BlockSpec((BN, 1), lambda i: (i, 0)),
            pl.BlockSpec((WPAD, H), lambda i: (0, 0)),
            pl.BlockSpec((H, H), lambda i: (0, 0)),
            pl.BlockSpec((H, H), lambda i: (0, 0)),
        ],
        out_specs=pl.BlockSpec((1, 1), lambda i: (0, 0)),
        out_shape=jax.ShapeDtypeStruct((1, 1), jnp.float32),
        scratch_shapes=[pltpu.VMEM((H, WPAD), jnp.float32)],
    )(hiddens, tgt2d, emb_pad, W_ih, W_hh)
    return out[0, 0]


# speedup vs baseline: 2.3023x; 2.3023x over previous
import jax
import jax.numpy as jnp
from jax.experimental import pallas as pl
from jax.experimental.pallas import tpu as pltpu

NTOK = 1000
SPLIT = 500
SENT = 1000
TEMP = 65.0
N = 1024
H = 64
WPAD = 1024
BN = 16
BW = 128
NCHUNK = WPAD // BW


def _loss_kernel(hid_ref, tgt_ref, emb_ref, wih_ref, whh_ref, out_ref,
                 at_ref):
    i = pl.program_id(0)

    @pl.when(i == 0)
    def _prep():
        a = jnp.dot(emb_ref[...], wih_ref[...],
                    preferred_element_type=jnp.float32)
        at_ref[...] = a.T
        out_ref[...] = jnp.zeros((1, 1), jnp.float32)

    h_blk = hid_ref[...]
    b_blk = jnp.dot(h_blk, whh_ref[...],
                    preferred_element_type=jnp.float32)
    tgt = tgt_ref[...]

    head_s = jnp.zeros((BN, 1), jnp.float32)
    tail_s = jnp.zeros((BN, 1), jnp.float32)
    tgt_k = jnp.zeros((BN, 1), jnp.float32)
    sent_k = jnp.zeros((BN, 1), jnp.float32)

    b3 = b_blk[:, :, None]
    h3 = h_blk[:, :, None]

    for c in range(NCHUNK):
        at_c = at_ref[:, c * BW:(c + 1) * BW]
        wids = c * BW + jax.lax.broadcasted_iota(
            jnp.int32, (1, BW), 1)
        t = jnp.tanh(at_c[None, :, :] + b3)
        diff = h3 - t
        d = jnp.sum(diff * diff, axis=1)
        k = TEMP / (1.0 + d)
        e = jnp.exp(k - TEMP)
        headmask = (wids < SPLIT) | (wids == SENT)
        tailmask = (wids >= SPLIT) & (wids < NTOK)
        head_s += jnp.sum(jnp.where(headmask, e, 0.0), axis=1,
                          keepdims=True)
        tail_s += jnp.sum(jnp.where(tailmask, e, 0.0), axis=1,
                          keepdims=True)
        tgt_k += jnp.sum(jnp.where(wids == tgt, k, 0.0), axis=1,
                         keepdims=True)
        sent_k += jnp.sum(jnp.where(wids == SENT, k, 0.0), axis=1,
                          keepdims=True)

    lse_head = jnp.log(head_s) + TEMP
    lse_tail = jnp.log(tail_s) + TEMP
    is_tail = (tgt >= SPLIT).astype(jnp.float32)
    loss = lse_head - tgt_k + is_tail * (lse_tail - sent_k)
    out_ref[...] += jnp.sum(loss, keepdims=True).reshape(1, 1) * (1.0 / N)


@jax.jit
def kernel(hiddens, targets, emb, W_ih, W_hh):
    emb_pad = jnp.zeros((WPAD, H), jnp.float32).at[:emb.shape[0]].set(emb)
    tgt2d = targets.reshape(N, 1)
    out = pl.pallas_call(
        _loss_kernel,
        grid=(N // BN,),
        in_specs=[
            pl.BlockSpec((BN, H), lambda i: (i, 0)),
            pl.BlockSpec((BN, 1), lambda i: (i, 0)),
            pl.BlockSpec((WPAD, H), lambda i: (0, 0)),
            pl.BlockSpec((H, H), lambda i: (0, 0)),
            pl.BlockSpec((H, H), lambda i: (0, 0)),
        ],
        out_specs=pl.BlockSpec((1, 1), lambda i: (0, 0)),
        out_shape=jax.ShapeDtypeStruct((1, 1), jnp.float32),
        scratch_shapes=[pltpu.VMEM((H, WPAD), jnp.float32)],
    )(hiddens, tgt2d, emb_pad, W_ih, W_hh)
    return out[0, 0]
